# Initial kernel scaffold; baseline (speedup 1.0000x reference)
import jax
import jax.numpy as jnp
from jax.experimental import pallas as pl


def kernel(semantic_map, W_embed, ln_gamma, ln_beta):
    raise NotImplementedError("write your pallas kernel here")


# speedup vs baseline: 56.6705x; 56.6705x over previous
import functools

import jax
import jax.numpy as jnp
from jax import lax
from jax.experimental import pallas as pl
from jax.experimental.pallas import tpu as pltpu
from jax.experimental.pallas import tpu_sc as plsc

_NUM_CLASSES = 256
_EMBED_DIM = 1024
_PATCH = 16

_B = 2
_H = 512
_W = 512
_HP = _H // _PATCH
_WP = _W // _PATCH
_NPATCH = _B * _HP * _WP
_PPP = _PATCH * _PATCH

_NC = 2
_NS = 16
_NW = _NC * _NS
_PATCH_PER_W = _NPATCH // _NW
_PIX_PER_W = _PATCH_PER_W * _PPP
_CHUNKS = _PIX_PER_W // 16


def _sc_histogram(idx_flat):
    mesh = plsc.VectorSubcoreMesh(core_axis_name="c", subcore_axis_name="s")

    @functools.partial(
        pl.kernel,
        mesh=mesh,
        out_type=jax.ShapeDtypeStruct((_NPATCH * _NUM_CLASSES,), jnp.float32),
        scratch_types=[
            pltpu.VMEM((_PIX_PER_W,), jnp.int32),
            pltpu.VMEM((_PATCH_PER_W * _NUM_CLASSES,), jnp.float32),
        ],
        compiler_params=pltpu.CompilerParams(needs_layout_passes=False),
    )
    def hist_kernel(idx_hbm, out_hbm, idx_v, cnt_v):
        wid = lax.axis_index("s") * _NC + lax.axis_index("c")
        pix_base = wid * _PIX_PER_W

        pltpu.sync_copy(idx_hbm.at[pl.ds(pix_base, _PIX_PER_W)], idx_v)

        zeros16 = jnp.zeros((16,), jnp.float32)

        def zero_body(k, _):
            cnt_v[pl.ds(k * 16, 16)] = zeros16
            return 0

        lax.fori_loop(0, _CHUNKS, zero_body, 0)

        ones16 = jnp.ones((16,), jnp.float32)

        def scatter_body(i, _):
            v = idx_v[pl.ds(i * 16, 16)]
            v = jnp.minimum(jnp.maximum(v, 0), _NUM_CLASSES - 1)
            p_local = (i // 512) * 32 + (i % 32)
            offs = v + p_local * _NUM_CLASSES
            plsc.addupdate_scatter(cnt_v, [offs], ones16)
            return 0

        lax.fori_loop(0, _CHUNKS, scatter_body, 0)

        out_base = wid * _PATCH_PER_W * _NUM_CLASSES
        pltpu.sync_copy(cnt_v, out_hbm.at[pl.ds(out_base, _PATCH_PER_W * _NUM_CLASSES)])

    return hist_kernel(idx_flat)


def _pos_embed_1d(length, d_half):
    p = jnp.arange(length, dtype=jnp.float32)
    om = jnp.arange(d_half, dtype=jnp.float32)
    om = 1.0 / 10000 ** (om / d_half)
    out = jnp.einsum("n,d->nd", p, om)
    return jnp.concatenate([jnp.sin(out), jnp.cos(out)], axis=1)


def _pos_embed_2d(h, w, dim):
    d_h = dim // 2
    d_w = dim - d_h
    d_h_even = (d_h // 2) * 2
    d_w_even = (d_w // 2) * 2
    emb_h = _pos_embed_1d(h, d_h_even // 2)
    emb_w = _pos_embed_1d(w, d_w_even // 2)
    emb_h = jnp.broadcast_to(emb_h[:, None, :], (h, w, d_h_even))
    emb_w = jnp.broadcast_to(emb_w[None, :, :], (h, w, d_w_even))
    pos = jnp.concatenate([emb_h, emb_w], axis=-1)
    if pos.shape[-1] < dim:
        pad = jnp.zeros((h, w, dim - pos.shape[-1]), dtype=jnp.float32)
        pos = jnp.concatenate([pos, pad], axis=-1)
    elif pos.shape[-1] > dim:
        pos = pos[:, :, :dim]
    return pos.reshape(h * w, dim)


_TOK_BLK = 256


def _tc_body(cnt_ref, w_ref, pos_ref, g_ref, b_ref, out_ref):
    x = jnp.dot(cnt_ref[...], w_ref[...], preferred_element_type=jnp.float32)
    x = x * (1.0 / _PPP) + pos_ref[...]
    mu = jnp.mean(x, axis=1, keepdims=True)
    xc = x - mu
    var = jnp.mean(xc * xc, axis=1, keepdims=True)
    out_ref[...] = xc * lax.rsqrt(var + 1e-5) * g_ref[...] + b_ref[...]


def _tc_pool_ln(counts, W_embed, pos, gamma, beta):
    nblk = _NPATCH // _TOK_BLK
    pos_blocks = _HP * _WP // _TOK_BLK
    return pl.pallas_call(
        _tc_body,
        grid=(nblk,),
        in_specs=[
            pl.BlockSpec((_TOK_BLK, _NUM_CLASSES), lambda j: (j, 0)),
            pl.BlockSpec((_NUM_CLASSES, _EMBED_DIM), lambda j: (0, 0)),
            pl.BlockSpec((_TOK_BLK, _EMBED_DIM), lambda j: (j % pos_blocks, 0)),
            pl.BlockSpec((1, _EMBED_DIM), lambda j: (0, 0)),
            pl.BlockSpec((1, _EMBED_DIM), lambda j: (0, 0)),
        ],
        out_specs=pl.BlockSpec((_TOK_BLK, _EMBED_DIM), lambda j: (j, 0)),
        out_shape=jax.ShapeDtypeStruct((_NPATCH, _EMBED_DIM), jnp.float32),
    )(counts, W_embed, pos, gamma, beta)


def kernel(semantic_map, W_embed, ln_gamma, ln_beta):
    idx_flat = semantic_map[:, 0].reshape(-1).astype(jnp.int32)
    counts = _sc_histogram(idx_flat).reshape(_NPATCH, _NUM_CLASSES)
    pos = _pos_embed_2d(_HP, _WP, _EMBED_DIM)
    tokens = _tc_pool_ln(
        counts,
        W_embed.astype(jnp.float32),
        pos,
        ln_gamma.reshape(1, _EMBED_DIM),
        ln_beta.reshape(1, _EMBED_DIM),
    )
    return tokens.reshape(_B, _HP * _WP, _EMBED_DIM)


# speedup vs baseline: 67.6736x; 1.1942x over previous
import functools

import jax
import jax.numpy as jnp
from jax import lax
from jax.experimental import pallas as pl
from jax.experimental.pallas import tpu as pltpu
from jax.experimental.pallas import tpu_sc as plsc

_NUM_CLASSES = 256
_EMBED_DIM = 1024
_PATCH = 16

_B = 2
_H = 512
_W = 512
_HP = _H // _PATCH
_WP = _W // _PATCH
_NPATCH = _B * _HP * _WP
_PPP = _PATCH * _PATCH

_NC = 2
_NS = 16
_NW = _NC * _NS
_PATCH_PER_W = _NPATCH // _NW
_ROWS_PER_W = _H * _B // _NW


def _sc_histogram(semantic_map):
    mesh = plsc.VectorSubcoreMesh(core_axis_name="c", subcore_axis_name="s")

    @functools.partial(
        pl.kernel,
        mesh=mesh,
        out_type=jax.ShapeDtypeStruct((_NPATCH, _NUM_CLASSES), jnp.float32),
        scratch_types=[
            pltpu.VMEM((_ROWS_PER_W, _W), jnp.int32),
            pltpu.VMEM((_PATCH_PER_W, _NUM_CLASSES), jnp.float32),
        ],
        compiler_params=pltpu.CompilerParams(needs_layout_passes=False),
    )
    def hist_kernel(sm_hbm, out_hbm, idx_v, cnt_v):
        wid = lax.axis_index("s") * _NC + lax.axis_index("c")
        b = wid // (_NS * _NC // _B)
        row0 = (wid % (_NS * _NC // _B)) * _ROWS_PER_W

        pltpu.sync_copy(sm_hbm.at[b, 0, pl.ds(row0, _ROWS_PER_W), :], idx_v)

        zeros16 = jnp.zeros((16,), jnp.float32)

        def zero_body(p, _):
            for t in range(_NUM_CLASSES // 16):
                cnt_v[p, pl.ds(t * 16, 16)] = zeros16
            return 0

        lax.fori_loop(0, _PATCH_PER_W, zero_body, 0)

        ones16 = jnp.ones((16,), jnp.float32)

        def scatter_body(r, _):
            prow = (r // _PATCH) * _WP
            for pw in range(_WP):
                v = idx_v[r, pl.ds(pw * 16, 16)]
                v = jnp.minimum(jnp.maximum(v, 0), _NUM_CLASSES - 1)
                rows = jnp.full((16,), prow + pw, jnp.int32)
                plsc.addupdate_scatter(cnt_v, [rows, v], ones16)
            return 0

        lax.fori_loop(0, _ROWS_PER_W, scatter_body, 0)

        pltpu.sync_copy(cnt_v, out_hbm.at[pl.ds(wid * _PATCH_PER_W, _PATCH_PER_W), :])

    return hist_kernel(semantic_map)


def _pos_embed_1d(length, d_half):
    p = jnp.arange(length, dtype=jnp.float32)
    om = jnp.arange(d_half, dtype=jnp.float32)
    om = 1.0 / 10000 ** (om / d_half)
    out = jnp.einsum("n,d->nd", p, om)
    return jnp.concatenate([jnp.sin(out), jnp.cos(out)], axis=1)


_TOK_BLK = 256
_PH_BLK = _TOK_BLK // _WP


def _tc_body(cnt_ref, w_ref, eh_ref, ew_ref, g_ref, b_ref, out_ref):
    x = jnp.dot(cnt_ref[...], w_ref[...], preferred_element_type=jnp.float32)
    eh = jnp.broadcast_to(
        eh_ref[...][:, None, :], (_PH_BLK, _WP, _EMBED_DIM // 2)
    ).reshape(_TOK_BLK, _EMBED_DIM // 2)
    ew = jnp.broadcast_to(
        ew_ref[...][None, :, :], (_PH_BLK, _WP, _EMBED_DIM // 2)
    ).reshape(_TOK_BLK, _EMBED_DIM // 2)
    pos = jnp.concatenate([eh, ew], axis=-1)
    x = x * (1.0 / _PPP) + pos
    mu = jnp.mean(x, axis=1, keepdims=True)
    xc = x - mu
    var = jnp.mean(xc * xc, axis=1, keepdims=True)
    out_ref[0] = xc * lax.rsqrt(var + 1e-5) * g_ref[...] + b_ref[...]


def _tc_pool_ln(counts, W_embed, emb_h, emb_w, gamma, beta):
    nblk = _HP // _PH_BLK
    return pl.pallas_call(
        _tc_body,
        grid=(_B, nblk),
        in_specs=[
            pl.BlockSpec((_TOK_BLK, _NUM_CLASSES), lambda b, j: (b * 4 + j, 0)),
            pl.BlockSpec((_NUM_CLASSES, _EMBED_DIM), lambda b, j: (0, 0)),
            pl.BlockSpec((_PH_BLK, _EMBED_DIM // 2), lambda b, j: (j, 0)),
            pl.BlockSpec((_WP, _EMBED_DIM // 2), lambda b, j: (0, 0)),
            pl.BlockSpec((1, _EMBED_DIM), lambda b, j: (0, 0)),
            pl.BlockSpec((1, _EMBED_DIM), lambda b, j: (0, 0)),
        ],
        out_specs=pl.BlockSpec((1, _TOK_BLK, _EMBED_DIM), lambda b, j: (b, j, 0)),
        out_shape=jax.ShapeDtypeStruct((_B, _HP * _WP, _EMBED_DIM), jnp.float32),
    )(counts, W_embed, emb_h, emb_w, gamma, beta)


def kernel(semantic_map, W_embed, ln_gamma, ln_beta):
    counts = _sc_histogram(semantic_map.astype(jnp.int32))
    emb_h = _pos_embed_1d(_HP, _EMBED_DIM // 4)
    emb_w = _pos_embed_1d(_WP, _EMBED_DIM // 4)
    return _tc_pool_ln(
        counts,
        W_embed.astype(jnp.float32),
        emb_h,
        emb_w,
        ln_gamma.reshape(1, _EMBED_DIM),
        ln_beta.reshape(1, _EMBED_DIM),
    )


# speedup vs baseline: 83.6547x; 1.2361x over previous
import functools

import jax
import jax.numpy as jnp
from jax import lax
from jax.experimental import pallas as pl
from jax.experimental.pallas import tpu as pltpu
from jax.experimental.pallas import tpu_sc as plsc

_NUM_CLASSES = 256
_EMBED_DIM = 1024
_PATCH = 16

_B = 2
_H = 512
_W = 512
_HP = _H // _PATCH
_WP = _W // _PATCH
_NPATCH = _B * _HP * _WP
_PPP = _PATCH * _PATCH

_NC = 2
_NS = 16
_NW = _NC * _NS
_PATCH_PER_W = _NPATCH // _NW
_ROWS_PER_W = _H * _B // _NW


def _sc_histogram(semantic_map):
    mesh = plsc.VectorSubcoreMesh(core_axis_name="c", subcore_axis_name="s")

    nbins = _PATCH_PER_W * _NUM_CLASSES
    nchunks = _ROWS_PER_W * _W // 16

    @functools.partial(
        pl.kernel,
        mesh=mesh,
        out_type=jax.ShapeDtypeStruct((_NPATCH * _NUM_CLASSES,), jnp.float32),
        scratch_types=[
            pltpu.VMEM((_ROWS_PER_W, _W), jnp.int32),
            pltpu.VMEM((nbins,), jnp.float32),
        ],
        compiler_params=pltpu.CompilerParams(needs_layout_passes=False),
    )
    def hist_kernel(sm_hbm, out_hbm, idx_v, cnt_v):
        wid = lax.axis_index("s") * _NC + lax.axis_index("c")
        b = wid // (_NS * _NC // _B)
        row0 = (wid % (_NS * _NC // _B)) * _ROWS_PER_W

        pltpu.sync_copy(sm_hbm.at[b, 0, pl.ds(row0, _ROWS_PER_W), :], idx_v)

        zeros16 = jnp.zeros((16,), jnp.float32)

        @plsc.parallel_loop(0, nbins // 16, 1, unroll=8)
        def _(k):
            cnt_v[pl.ds(k * 16, 16)] = zeros16

        ones16 = jnp.ones((16,), jnp.float32)

        @plsc.parallel_loop(0, nchunks, 1, unroll=8)
        def _(i):
            v = idx_v[i // 32, pl.ds((i % 32) * 16, 16)]
            base = ((i // 512) * 32 + (i % 32)) * _NUM_CLASSES
            plsc.addupdate_scatter(cnt_v, [v + base], ones16)

        pltpu.sync_copy(cnt_v, out_hbm.at[pl.ds(wid * nbins, nbins)])

    return hist_kernel(semantic_map).reshape(_NPATCH, _NUM_CLASSES)


def _pos_embed_1d(length, d_half):
    p = jnp.arange(length, dtype=jnp.float32)
    om = jnp.arange(d_half, dtype=jnp.float32)
    om = 1.0 / 10000 ** (om / d_half)
    out = jnp.einsum("n,d->nd", p, om)
    return jnp.concatenate([jnp.sin(out), jnp.cos(out)], axis=1)


_TOK_BLK = 512
_PH_BLK = _TOK_BLK // _WP


def _tc_body(cnt_ref, w_ref, eh_ref, ew_ref, g_ref, b_ref, out_ref):
    x = jnp.dot(cnt_ref[...], w_ref[...], preferred_element_type=jnp.float32)
    eh = jnp.broadcast_to(
        eh_ref[...][:, None, :], (_PH_BLK, _WP, _EMBED_DIM // 2)
    ).reshape(_TOK_BLK, _EMBED_DIM // 2)
    ew = jnp.broadcast_to(
        ew_ref[...][None, :, :], (_PH_BLK, _WP, _EMBED_DIM // 2)
    ).reshape(_TOK_BLK, _EMBED_DIM // 2)
    pos = jnp.concatenate([eh, ew], axis=-1)
    x = x * (1.0 / _PPP) + pos
    mu = jnp.mean(x, axis=1, keepdims=True)
    xc = x - mu
    var = jnp.mean(xc * xc, axis=1, keepdims=True)
    out_ref[0] = xc * lax.rsqrt(var + 1e-5) * g_ref[...] + b_ref[...]


def _tc_pool_ln(counts, W_embed, emb_h, emb_w, gamma, beta):
    nblk = _HP // _PH_BLK
    return pl.pallas_call(
        _tc_body,
        grid=(_B, nblk),
        in_specs=[
            pl.BlockSpec((_TOK_BLK, _NUM_CLASSES), lambda b, j: (b * 2 + j, 0)),
            pl.BlockSpec((_NUM_CLASSES, _EMBED_DIM), lambda b, j: (0, 0)),
            pl.BlockSpec((_PH_BLK, _EMBED_DIM // 2), lambda b, j: (j, 0)),
            pl.BlockSpec((_WP, _EMBED_DIM // 2), lambda b, j: (0, 0)),
            pl.BlockSpec((1, _EMBED_DIM), lambda b, j: (0, 0)),
            pl.BlockSpec((1, _EMBED_DIM), lambda b, j: (0, 0)),
        ],
        out_specs=pl.BlockSpec((1, _TOK_BLK, _EMBED_DIM), lambda b, j: (b, j, 0)),
        out_shape=jax.ShapeDtypeStruct((_B, _HP * _WP, _EMBED_DIM), jnp.float32),
    )(counts, W_embed, emb_h, emb_w, gamma, beta)


def kernel(semantic_map, W_embed, ln_gamma, ln_beta):
    counts = _sc_histogram(semantic_map.astype(jnp.int32))
    emb_h = _pos_embed_1d(_HP, _EMBED_DIM // 4)
    emb_w = _pos_embed_1d(_WP, _EMBED_DIM // 4)
    return _tc_pool_ln(
        counts,
        W_embed.astype(jnp.float32),
        emb_h,
        emb_w,
        ln_gamma.reshape(1, _EMBED_DIM),
        ln_beta.reshape(1, _EMBED_DIM),
    )


# speedup vs baseline: 91.1339x; 1.0894x over previous
import functools

import numpy as np

import jax
import jax.numpy as jnp
from jax import lax
from jax.experimental import pallas as pl
from jax.experimental.pallas import tpu as pltpu
from jax.experimental.pallas import tpu_sc as plsc

_NUM_CLASSES = 256
_EMBED_DIM = 1024
_PATCH = 16

_B = 2
_H = 512
_W = 512
_HP = _H // _PATCH
_WP = _W // _PATCH
_NPATCH = _B * _HP * _WP
_PPP = _PATCH * _PATCH

_NC = 2
_NS = 16
_NW = _NC * _NS
_PATCH_PER_W = _NPATCH // _NW
_ROWS_PER_W = _H * _B // _NW


def _sc_histogram(semantic_map):
    mesh = plsc.VectorSubcoreMesh(core_axis_name="c", subcore_axis_name="s")

    nbins = _PATCH_PER_W * _NUM_CLASSES
    nchunks = _ROWS_PER_W * _W // 16

    @functools.partial(
        pl.kernel,
        mesh=mesh,
        out_type=jax.ShapeDtypeStruct((_NPATCH, _NUM_CLASSES), jnp.float32),
        scratch_types=[
            pltpu.VMEM((_ROWS_PER_W, _W), jnp.int32),
            pltpu.VMEM((_PATCH_PER_W, _NUM_CLASSES), jnp.float32),
        ],
        compiler_params=pltpu.CompilerParams(
            needs_layout_passes=False, skip_device_barrier=True
        ),
    )
    def hist_kernel(sm_hbm, out_hbm, idx_v, cnt_v):
        wid = lax.axis_index("s") * _NC + lax.axis_index("c")
        b = wid // (_NS * _NC // _B)
        row0 = (wid % (_NS * _NC // _B)) * _ROWS_PER_W

        pltpu.sync_copy(sm_hbm.at[b, 0, pl.ds(row0, _ROWS_PER_W), :], idx_v)

        zeros16 = jnp.zeros((16,), jnp.float32)

        @plsc.parallel_loop(0, nbins // 16, 1, unroll=8)
        def _(k):
            cnt_v[k // 16, pl.ds((k % 16) * 16, 16)] = zeros16

        ones16 = jnp.ones((16,), jnp.float32)

        @plsc.parallel_loop(0, nchunks, 1, unroll=8)
        def _(i):
            v = idx_v[i // 32, pl.ds((i % 32) * 16, 16)]
            p = jnp.full((16,), (i // 512) * 32 + (i % 32), jnp.int32)
            plsc.addupdate_scatter(cnt_v, [p, v], ones16)

        pltpu.sync_copy(cnt_v, out_hbm.at[pl.ds(wid * _PATCH_PER_W, _PATCH_PER_W), :])

    return hist_kernel(semantic_map)


def _pos_embed_1d(length, d_half):
    p = np.arange(length, dtype=np.float32)
    om = 1.0 / 10000 ** (np.arange(d_half, dtype=np.float32) / d_half)
    out = np.outer(p, om)
    return jnp.asarray(
        np.concatenate([np.sin(out), np.cos(out)], axis=1), dtype=jnp.float32
    )


_TOK_BLK = 512
_PH_BLK = _TOK_BLK // _WP


def _tc_body(cnt_ref, w_ref, eh_ref, ew_ref, g_ref, b_ref, out_ref):
    x = jnp.dot(cnt_ref[...], w_ref[...], preferred_element_type=jnp.float32)
    eh = jnp.broadcast_to(
        eh_ref[...][:, None, :], (_PH_BLK, _WP, _EMBED_DIM // 2)
    ).reshape(_TOK_BLK, _EMBED_DIM // 2)
    ew = jnp.broadcast_to(
        ew_ref[...][None, :, :], (_PH_BLK, _WP, _EMBED_DIM // 2)
    ).reshape(_TOK_BLK, _EMBED_DIM // 2)
    pos = jnp.concatenate([eh, ew], axis=-1)
    x = x * (1.0 / _PPP) + pos
    mu = jnp.mean(x, axis=1, keepdims=True)
    xc = x - mu
    var = jnp.mean(xc * xc, axis=1, keepdims=True)
    out_ref[0] = xc * lax.rsqrt(var + 1e-5) * g_ref[...] + b_ref[...]


def _tc_pool_ln(counts, W_embed, emb_h, emb_w, gamma, beta):
    nblk = _HP // _PH_BLK
    return pl.pallas_call(
        _tc_body,
        grid=(_B, nblk),
        in_specs=[
            pl.BlockSpec((_TOK_BLK, _NUM_CLASSES), lambda b, j: (b * 2 + j, 0)),
            pl.BlockSpec((_NUM_CLASSES, _EMBED_DIM), lambda b, j: (0, 0)),
            pl.BlockSpec((_PH_BLK, _EMBED_DIM // 2), lambda b, j: (j, 0)),
            pl.BlockSpec((_WP, _EMBED_DIM // 2), lambda b, j: (0, 0)),
            pl.BlockSpec((1, _EMBED_DIM), lambda b, j: (0, 0)),
            pl.BlockSpec((1, _EMBED_DIM), lambda b, j: (0, 0)),
        ],
        out_specs=pl.BlockSpec((1, _TOK_BLK, _EMBED_DIM), lambda b, j: (b, j, 0)),
        out_shape=jax.ShapeDtypeStruct((_B, _HP * _WP, _EMBED_DIM), jnp.float32),
    )(counts, W_embed, emb_h, emb_w, gamma, beta)


def kernel(semantic_map, W_embed, ln_gamma, ln_beta):
    counts = _sc_histogram(semantic_map.astype(jnp.int32))
    emb_h = _pos_embed_1d(_HP, _EMBED_DIM // 4)
    emb_w = _pos_embed_1d(_WP, _EMBED_DIM // 4)
    return _tc_pool_ln(
        counts,
        W_embed.astype(jnp.float32),
        emb_h,
        emb_w,
        ln_gamma.reshape(1, _EMBED_DIM),
        ln_beta.reshape(1, _EMBED_DIM),
    )


# speedup vs baseline: 92.6232x; 1.0163x over previous
import functools

import numpy as np

import jax
import jax.numpy as jnp
from jax import lax
from jax.experimental import pallas as pl
from jax.experimental.pallas import tpu as pltpu
from jax.experimental.pallas import tpu_sc as plsc

_NUM_CLASSES = 256
_EMBED_DIM = 1024
_PATCH = 16

_B = 2
_H = 512
_W = 512
_HP = _H // _PATCH
_WP = _W // _PATCH
_NPATCH = _B * _HP * _WP
_PPP = _PATCH * _PATCH

_NC = 2
_NS = 16
_NW = _NC * _NS
_PATCH_PER_W = _NPATCH // _NW
_ROWS_PER_W = _H * _B // _NW


def _sc_histogram(semantic_map):
    mesh = plsc.VectorSubcoreMesh(core_axis_name="c", subcore_axis_name="s")

    nbins = _PATCH_PER_W * _NUM_CLASSES
    nchunks = _ROWS_PER_W * _W // 16

    @functools.partial(
        pl.kernel,
        mesh=mesh,
        out_type=jax.ShapeDtypeStruct((_NPATCH, _NUM_CLASSES), jnp.float32),
        scratch_types=[
            pltpu.VMEM((_ROWS_PER_W, _W), jnp.int32),
            pltpu.VMEM((_PATCH_PER_W, _NUM_CLASSES), jnp.float32),
        ],
        compiler_params=pltpu.CompilerParams(
            needs_layout_passes=False, skip_device_barrier=True
        ),
    )
    def hist_kernel(sm_hbm, out_hbm, idx_v, cnt_v):
        wid = lax.axis_index("s") * _NC + lax.axis_index("c")
        b = wid // (_NS * _NC // _B)
        row0 = (wid % (_NS * _NC // _B)) * _ROWS_PER_W

        pltpu.sync_copy(sm_hbm.at[b, 0, pl.ds(row0, _ROWS_PER_W), :], idx_v)

        zeros16 = jnp.zeros((16,), jnp.float32)

        @plsc.parallel_loop(0, nbins // 16, 1, unroll=8)
        def _(k):
            cnt_v[k // 16, pl.ds((k % 16) * 16, 16)] = zeros16

        ones16 = jnp.ones((16,), jnp.float32)

        @plsc.parallel_loop(0, nchunks, 1, unroll=16)
        def _(i):
            v = idx_v[i // 32, pl.ds((i % 32) * 16, 16)]
            p = jnp.full((16,), (i // 512) * 32 + (i % 32), jnp.int32)
            plsc.addupdate_scatter(cnt_v, [p, v], ones16)

        pltpu.sync_copy(cnt_v, out_hbm.at[pl.ds(wid * _PATCH_PER_W, _PATCH_PER_W), :])

    return hist_kernel(semantic_map)


def _pos_embed_1d(length, d_half):
    p = np.arange(length, dtype=np.float32)
    om = 1.0 / 10000 ** (np.arange(d_half, dtype=np.float32) / d_half)
    out = np.outer(p, om)
    return jnp.asarray(
        np.concatenate([np.sin(out), np.cos(out)], axis=1), dtype=jnp.float32
    )


_TOK_BLK = 1024
_PH_BLK = _TOK_BLK // _WP


def _tc_body(cnt_ref, w_ref, eh_ref, ew_ref, g_ref, b_ref, out_ref):
    x = jnp.dot(cnt_ref[...], w_ref[...], preferred_element_type=jnp.float32)
    eh = jnp.broadcast_to(
        eh_ref[...][:, None, :], (_PH_BLK, _WP, _EMBED_DIM // 2)
    ).reshape(_TOK_BLK, _EMBED_DIM // 2)
    ew = jnp.broadcast_to(
        ew_ref[...][None, :, :], (_PH_BLK, _WP, _EMBED_DIM // 2)
    ).reshape(_TOK_BLK, _EMBED_DIM // 2)
    pos = jnp.concatenate([eh, ew], axis=-1)
    x = x * (1.0 / _PPP) + pos
    mu = jnp.mean(x, axis=1, keepdims=True)
    xc = x - mu
    var = jnp.mean(xc * xc, axis=1, keepdims=True)
    out_ref[0] = xc * lax.rsqrt(var + 1e-5) * g_ref[...] + b_ref[...]


def _tc_pool_ln(counts, W_embed, emb_h, emb_w, gamma, beta):
    nblk = _HP // _PH_BLK
    return pl.pallas_call(
        _tc_body,
        grid=(_B, nblk),
        compiler_params=pltpu.CompilerParams(skip_device_barrier=True),
        in_specs=[
            pl.BlockSpec((_TOK_BLK, _NUM_CLASSES), lambda b, j: (b * nblk + j, 0)),
            pl.BlockSpec((_NUM_CLASSES, _EMBED_DIM), lambda b, j: (0, 0)),
            pl.BlockSpec((_PH_BLK, _EMBED_DIM // 2), lambda b, j: (j, 0)),
            pl.BlockSpec((_WP, _EMBED_DIM // 2), lambda b, j: (0, 0)),
            pl.BlockSpec((1, _EMBED_DIM), lambda b, j: (0, 0)),
            pl.BlockSpec((1, _EMBED_DIM), lambda b, j: (0, 0)),
        ],
        out_specs=pl.BlockSpec((1, _TOK_BLK, _EMBED_DIM), lambda b, j: (b, j, 0)),
        out_shape=jax.ShapeDtypeStruct((_B, _HP * _WP, _EMBED_DIM), jnp.float32),
    )(counts, W_embed, emb_h, emb_w, gamma, beta)


def kernel(semantic_map, W_embed, ln_gamma, ln_beta):
    counts = _sc_histogram(semantic_map.astype(jnp.int32))
    emb_h = _pos_embed_1d(_HP, _EMBED_DIM // 4)
    emb_w = _pos_embed_1d(_WP, _EMBED_DIM // 4)
    return _tc_pool_ln(
        counts,
        W_embed.astype(jnp.float32),
        emb_h,
        emb_w,
        ln_gamma.reshape(1, _EMBED_DIM),
        ln_beta.reshape(1, _EMBED_DIM),
    )
